# baseline (speedup 1.0000x reference)
import functools

import jax
import jax.numpy as jnp
from jax.experimental import pallas as pl
from jax.experimental.pallas import tpu as pltpu
from jax.experimental.pallas import tpu_sc as plsc

_NET = [
    (0, 1, True, True), (2, 3, True, True), (4, 5, True, True), (6, 7, True, True), (8, 9, True, True), (10, 11, True, True), (12, 13, True, True), (14, 15, True, True),
    (16, 17, True, True), (18, 19, True, True), (20, 21, True, True), (22, 23, True, True), (24, 25, True, True), (26, 27, True, True), (28, 29, True, True), (30, 31, True, True),
    (0, 2, True, True), (1, 3, True, True), (4, 6, True, True), (5, 7, True, True), (8, 10, True, True), (9, 11, True, True), (12, 14, True, True), (13, 15, True, True),
    (16, 18, True, True), (17, 19, True, True), (20, 22, True, True), (21, 23, True, True), (24, 26, True, True), (25, 27, True, True), (28, 30, True, True), (29, 31, True, True),
    (1, 2, True, True), (5, 6, True, True), (9, 10, True, True), (13, 14, True, True), (17, 18, True, True), (21, 22, True, True), (25, 26, True, True), (29, 30, True, True),
    (0, 4, True, True), (1, 5, True, True), (2, 6, True, True), (3, 7, True, True), (8, 12, True, True), (9, 13, True, True), (10, 14, True, True), (11, 15, True, True),
    (16, 20, True, True), (17, 21, True, True), (18, 22, True, True), (19, 23, True, True), (24, 28, True, True), (25, 29, True, True), (26, 30, True, True), (27, 31, True, True),
    (2, 4, True, True), (3, 5, True, True), (10, 12, True, True), (11, 13, True, True), (18, 20, True, True), (19, 21, True, True), (26, 28, True, True), (27, 29, True, True),
    (1, 2, True, True), (3, 4, True, True), (5, 6, True, True), (9, 10, True, True), (11, 12, True, True), (13, 14, True, True), (17, 18, True, True), (19, 20, True, True),
    (21, 22, True, True), (25, 26, True, True), (27, 28, True, True), (29, 30, True, True), (0, 8, True, True), (1, 9, True, True), (2, 10, True, True), (3, 11, True, True),
    (4, 12, True, True), (5, 13, True, True), (6, 14, True, True), (7, 15, True, True), (16, 24, True, True), (17, 25, True, True), (18, 26, True, True), (19, 27, True, True),
    (20, 28, True, True), (21, 29, True, True), (22, 30, True, True), (23, 31, True, True), (4, 8, True, True), (5, 9, True, True), (6, 10, True, True), (7, 11, True, True),
    (20, 24, True, True), (21, 25, True, True), (22, 26, True, True), (23, 27, True, True), (2, 4, True, True), (3, 5, True, True), (6, 8, True, True), (7, 9, True, True),
    (10, 12, True, True), (11, 13, True, True), (18, 20, True, True), (19, 21, True, True), (22, 24, True, True), (23, 25, True, True), (26, 28, True, True), (27, 29, True, True),
    (1, 2, True, True), (3, 4, True, True), (5, 6, True, True), (7, 8, True, True), (9, 10, True, True), (11, 12, True, True), (13, 14, True, True), (17, 18, True, True),
    (19, 20, True, True), (21, 22, True, True), (23, 24, True, True), (25, 26, True, True), (27, 28, True, True), (29, 30, True, True), (0, 16, False, True), (1, 17, False, True),
    (2, 18, False, True), (3, 19, False, True), (4, 20, False, True), (5, 21, False, True), (6, 22, False, True), (7, 23, False, True), (8, 24, True, False), (9, 25, True, False),
    (10, 26, True, False), (11, 27, True, False), (12, 28, True, False), (13, 29, True, False), (14, 30, True, False), (15, 31, True, False), (8, 16, False, True), (9, 17, False, True),
    (10, 18, False, True), (11, 19, False, True), (12, 20, True, False), (13, 21, True, False), (14, 22, True, False), (15, 23, True, False), (12, 16, False, True), (13, 17, False, True),
    (14, 18, True, False), (15, 19, True, False), (14, 16, False, True), (15, 17, True, False), (15, 16, True, True),
]

_CHUNK = 128
_NCORES = 2
_NSUB = 16
_NW = _NCORES * _NSUB


_NBUF = 2


def _sc_gather(xT, idx):
    R = idx.shape[0]
    F = xT.shape[1]
    dt = xT.dtype
    n_chunks = R // _CHUNK
    pw = -(-n_chunks // _NW)
    n_outer = -(-pw // _NBUF)
    mesh = plsc.VectorSubcoreMesh(
        core_axis_name="c", subcore_axis_name="s",
        num_cores=_NCORES, num_subcores=_NSUB)

    n_rows = xT.shape[0]
    assert n_rows % (_NSUB * 8) == 0
    rows_per_sub = n_rows // _NSUB

    @functools.partial(
        pl.kernel,
        out_type=jax.ShapeDtypeStruct((R, F), dt),
        mesh=mesh,
        scratch_types=[
            pltpu.VMEM((_NBUF, _CHUNK), jnp.int32),
            pltpu.VMEM((_NBUF, _CHUNK, F), dt),
            pltpu.VMEM_SHARED((n_rows, F), dt),
            pltpu.SemaphoreType.DMA((_NBUF,)),
            pltpu.SemaphoreType.DMA((_NBUF,)),
            pltpu.SemaphoreType.DMA((_NBUF,)),
        ],
    )
    def gather_kernel(xT_hbm, idx_hbm, out_hbm, idx_v, rows_v, table_sp,
                      idx_sem, g_sem, wb_sem):
        cid = jax.lax.axis_index("c")
        sid = jax.lax.axis_index("s")
        wid = sid * _NCORES + cid
        c0 = wid * pw
        cnt = jnp.minimum(n_chunks - c0, pw)

        r0 = sid * rows_per_sub
        pltpu.sync_copy(xT_hbm.at[pl.ds(r0, rows_per_sub)],
                        table_sp.at[pl.ds(r0, rows_per_sub)])
        plsc.subcore_barrier()

        def idx_copy(i, b):
            return pltpu.make_async_copy(
                idx_hbm.at[pl.ds((c0 + i) * _CHUNK, _CHUNK)],
                idx_v.at[b], idx_sem.at[b])

        def wb_copy(i, b):
            return pltpu.make_async_copy(
                rows_v.at[b],
                out_hbm.at[pl.ds((c0 + i) * _CHUNK, _CHUNK)], wb_sem.at[b])

        for b in range(_NBUF):
            @pl.when(b < cnt)
            def _(b=b):
                idx_copy(b, b).start()

        @pl.loop(0, n_outer)
        def _(o):
            for b in range(_NBUF):
                i = o * _NBUF + b
                pb = (b - 1) % _NBUF

                @pl.when(i < cnt)
                def _(i=i, b=b, pb=pb):
                    @pl.when(i >= _NBUF)
                    def _():
                        wb_copy(i - _NBUF, b).wait()

                    idx_copy(i, b).wait()
                    pltpu.make_async_copy(
                        table_sp.at[idx_v.at[b]], rows_v.at[b],
                        g_sem.at[b]).start()

                    @pl.when(i >= 1)
                    def _():
                        pltpu.make_async_copy(
                            table_sp.at[idx_v.at[pb]], rows_v.at[pb],
                            g_sem.at[pb]).wait()
                        wb_copy(i - 1, pb).start()

                        @pl.when(i - 1 + _NBUF < cnt)
                        def _():
                            idx_copy(i - 1 + _NBUF, pb).start()

        for b in range(_NBUF):
            @pl.when((cnt > 0) & ((cnt - 1) % _NBUF == b))
            def _(b=b):
                pltpu.make_async_copy(
                    table_sp.at[idx_v.at[b]], rows_v.at[b], g_sem.at[b]).wait()
                wb_copy(cnt - 1, b).start()

        for b in range(_NBUF):
            @pl.when(b < cnt)
            def _(b=b):
                wb_copy(0, b).wait()

    return gather_kernel(xT, idx)


def _tc_median_body(w_ref, xg_ref, x_ref, out_ref, med_ref):
    nb = med_ref.shape[0]

    def sub(s, carry):
        row = pl.multiple_of(s * 8, 8)
        v = {}
        for a in range(32):
            v[a] = xg_ref[1 + a, pl.ds(row, 8), :]
        for (a, b, need_min, need_max) in _NET:
            va, vb = v[a], v[b]
            if need_min:
                lo = jnp.minimum(va, vb)
            if need_max:
                v[b] = jnp.maximum(va, vb)
            if need_min:
                v[a] = lo
        slf = xg_ref[0, pl.ds(row, 8), :]
        med = jnp.minimum(jnp.maximum(slf, v[15]), v[16])
        med_ref[pl.ds(row, 8), :] = med
        return carry

    jax.lax.fori_loop(0, nb // 8, sub, 0)

    w0 = w_ref[0, 0]
    w1 = w_ref[0, 1]
    med = med_ref[...]
    eye = (jax.lax.broadcasted_iota(jnp.int32, (nb, nb), 0) ==
           jax.lax.broadcasted_iota(jnp.int32, (nb, nb), 1)).astype(med.dtype)
    medT = jax.lax.dot_general(med, eye, (((0,), (0,)), ((), ())),
                               preferred_element_type=jnp.float32)
    out_ref[0, :, :] = w0 * x_ref[0, :, :] + w1 * medT


def _tc_median(xg3, x, weight, interpret=False):
    m, n_pad, F = xg3.shape
    B, _, N = x.shape
    NB = _NB
    n_blocks = n_pad // NB
    return pl.pallas_call(
        _tc_median_body,
        grid=(n_blocks,),
        in_specs=[
            pl.BlockSpec(memory_space=pltpu.SMEM),
            pl.BlockSpec((m, NB, F), lambda i: (0, i, 0)),
            pl.BlockSpec((1, F, NB), lambda i: (0, 0, i)),
        ],
        out_specs=pl.BlockSpec((1, F, NB), lambda i: (0, 0, i)),
        out_shape=jax.ShapeDtypeStruct((B, F, N), jnp.float32),
        scratch_shapes=[pltpu.VMEM((NB, F), xg3.dtype)],
        interpret=interpret,
    )(weight, xg3, x)


_NB = 256
_K = 2


def kernel(x, neighbors, weight):
    B, F, N = x.shape
    NB = _NB
    n_blocks = -(-N // NB)
    n_pad = n_blocks * NB
    xT = jnp.pad(x[0].T, ((0, n_pad - N), (0, 0)))
    idx = jnp.concatenate(
        [jnp.arange(N, dtype=jnp.int32)[None, :], neighbors.T.astype(jnp.int32)],
        axis=0)
    idx = jnp.pad(idx, ((0, 0), (0, n_pad - N)))
    blocks_per_chunk = -(-n_blocks // _K)
    span = blocks_per_chunk * NB
    outs = []
    for k in range(_K):
        lo = k * span
        sz = min(span, n_pad - lo)
        if sz <= 0:
            break
        idx_k = idx[:, lo:lo + sz].reshape(-1)
        xg = _sc_gather(xT, idx_k)
        xg3 = xg.reshape(idx_k.shape[0] // sz, sz, F)
        x_k = jax.lax.dynamic_slice_in_dim(x, lo, min(sz, N - lo), axis=2)
        outs.append(_tc_median(xg3, x_k, weight))
    return jnp.concatenate(outs, axis=2) if len(outs) > 1 else outs[0]


# speedup vs baseline: 1.0340x; 1.0340x over previous
import functools

import jax
import jax.numpy as jnp
from jax.experimental import pallas as pl
from jax.experimental.pallas import tpu as pltpu
from jax.experimental.pallas import tpu_sc as plsc

_NET = [
    (0, 1, True, True), (2, 3, True, True), (4, 5, True, True), (6, 7, True, True), (8, 9, True, True), (10, 11, True, True), (12, 13, True, True), (14, 15, True, True),
    (16, 17, True, True), (18, 19, True, True), (20, 21, True, True), (22, 23, True, True), (24, 25, True, True), (26, 27, True, True), (28, 29, True, True), (30, 31, True, True),
    (0, 2, True, True), (1, 3, True, True), (4, 6, True, True), (5, 7, True, True), (8, 10, True, True), (9, 11, True, True), (12, 14, True, True), (13, 15, True, True),
    (16, 18, True, True), (17, 19, True, True), (20, 22, True, True), (21, 23, True, True), (24, 26, True, True), (25, 27, True, True), (28, 30, True, True), (29, 31, True, True),
    (1, 2, True, True), (5, 6, True, True), (9, 10, True, True), (13, 14, True, True), (17, 18, True, True), (21, 22, True, True), (25, 26, True, True), (29, 30, True, True),
    (0, 4, True, True), (1, 5, True, True), (2, 6, True, True), (3, 7, True, True), (8, 12, True, True), (9, 13, True, True), (10, 14, True, True), (11, 15, True, True),
    (16, 20, True, True), (17, 21, True, True), (18, 22, True, True), (19, 23, True, True), (24, 28, True, True), (25, 29, True, True), (26, 30, True, True), (27, 31, True, True),
    (2, 4, True, True), (3, 5, True, True), (10, 12, True, True), (11, 13, True, True), (18, 20, True, True), (19, 21, True, True), (26, 28, True, True), (27, 29, True, True),
    (1, 2, True, True), (3, 4, True, True), (5, 6, True, True), (9, 10, True, True), (11, 12, True, True), (13, 14, True, True), (17, 18, True, True), (19, 20, True, True),
    (21, 22, True, True), (25, 26, True, True), (27, 28, True, True), (29, 30, True, True), (0, 8, True, True), (1, 9, True, True), (2, 10, True, True), (3, 11, True, True),
    (4, 12, True, True), (5, 13, True, True), (6, 14, True, True), (7, 15, True, True), (16, 24, True, True), (17, 25, True, True), (18, 26, True, True), (19, 27, True, True),
    (20, 28, True, True), (21, 29, True, True), (22, 30, True, True), (23, 31, True, True), (4, 8, True, True), (5, 9, True, True), (6, 10, True, True), (7, 11, True, True),
    (20, 24, True, True), (21, 25, True, True), (22, 26, True, True), (23, 27, True, True), (2, 4, True, True), (3, 5, True, True), (6, 8, True, True), (7, 9, True, True),
    (10, 12, True, True), (11, 13, True, True), (18, 20, True, True), (19, 21, True, True), (22, 24, True, True), (23, 25, True, True), (26, 28, True, True), (27, 29, True, True),
    (1, 2, True, True), (3, 4, True, True), (5, 6, True, True), (7, 8, True, True), (9, 10, True, True), (11, 12, True, True), (13, 14, True, True), (17, 18, True, True),
    (19, 20, True, True), (21, 22, True, True), (23, 24, True, True), (25, 26, True, True), (27, 28, True, True), (29, 30, True, True), (0, 16, False, True), (1, 17, False, True),
    (2, 18, False, True), (3, 19, False, True), (4, 20, False, True), (5, 21, False, True), (6, 22, False, True), (7, 23, False, True), (8, 24, True, False), (9, 25, True, False),
    (10, 26, True, False), (11, 27, True, False), (12, 28, True, False), (13, 29, True, False), (14, 30, True, False), (15, 31, True, False), (8, 16, False, True), (9, 17, False, True),
    (10, 18, False, True), (11, 19, False, True), (12, 20, True, False), (13, 21, True, False), (14, 22, True, False), (15, 23, True, False), (12, 16, False, True), (13, 17, False, True),
    (14, 18, True, False), (15, 19, True, False), (14, 16, False, True), (15, 17, True, False), (15, 16, True, True),
]

_CHUNK = 128
_NCORES = 2
_NSUB = 16
_NW = _NCORES * _NSUB


_NBUF = 2


def _sc_gather(xT, idx):
    R = idx.shape[0]
    F = xT.shape[1]
    dt = xT.dtype
    n_chunks = R // _CHUNK
    pw = -(-n_chunks // _NW)
    n_outer = -(-pw // _NBUF)
    mesh = plsc.VectorSubcoreMesh(
        core_axis_name="c", subcore_axis_name="s",
        num_cores=_NCORES, num_subcores=_NSUB)

    n_rows = xT.shape[0]
    assert n_rows % (_NSUB * 8) == 0
    rows_per_sub = n_rows // _NSUB

    @functools.partial(
        pl.kernel,
        out_type=jax.ShapeDtypeStruct((R, F), dt),
        mesh=mesh,
        scratch_types=[
            pltpu.VMEM((_NBUF, _CHUNK), jnp.int32),
            pltpu.VMEM((_NBUF, _CHUNK, F), dt),
            pltpu.VMEM_SHARED((n_rows, F), dt),
            pltpu.SemaphoreType.DMA((_NBUF,)),
            pltpu.SemaphoreType.DMA((_NBUF,)),
            pltpu.SemaphoreType.DMA((_NBUF,)),
        ],
    )
    def gather_kernel(xT_hbm, idx_hbm, out_hbm, idx_v, rows_v, table_sp,
                      idx_sem, g_sem, wb_sem):
        cid = jax.lax.axis_index("c")
        sid = jax.lax.axis_index("s")
        wid = sid * _NCORES + cid
        c0 = wid * pw
        cnt = jnp.minimum(n_chunks - c0, pw)

        r0 = sid * rows_per_sub
        pltpu.sync_copy(xT_hbm.at[pl.ds(r0, rows_per_sub)],
                        table_sp.at[pl.ds(r0, rows_per_sub)])
        plsc.subcore_barrier()

        def idx_copy(i, b):
            return pltpu.make_async_copy(
                idx_hbm.at[pl.ds((c0 + i) * _CHUNK, _CHUNK)],
                idx_v.at[b], idx_sem.at[b])

        def wb_copy(i, b):
            return pltpu.make_async_copy(
                rows_v.at[b],
                out_hbm.at[pl.ds((c0 + i) * _CHUNK, _CHUNK)], wb_sem.at[b])

        for b in range(_NBUF):
            @pl.when(b < cnt)
            def _(b=b):
                idx_copy(b, b).start()

        @pl.loop(0, n_outer)
        def _(o):
            for b in range(_NBUF):
                i = o * _NBUF + b
                pb = (b - 1) % _NBUF

                @pl.when(i < cnt)
                def _(i=i, b=b, pb=pb):
                    @pl.when(i >= _NBUF)
                    def _():
                        wb_copy(i - _NBUF, b).wait()

                    idx_copy(i, b).wait()
                    pltpu.make_async_copy(
                        table_sp.at[idx_v.at[b]], rows_v.at[b],
                        g_sem.at[b]).start()

                    @pl.when(i >= 1)
                    def _():
                        pltpu.make_async_copy(
                            table_sp.at[idx_v.at[pb]], rows_v.at[pb],
                            g_sem.at[pb]).wait()
                        wb_copy(i - 1, pb).start()

                        @pl.when(i - 1 + _NBUF < cnt)
                        def _():
                            idx_copy(i - 1 + _NBUF, pb).start()

        for b in range(_NBUF):
            @pl.when((cnt > 0) & ((cnt - 1) % _NBUF == b))
            def _(b=b):
                pltpu.make_async_copy(
                    table_sp.at[idx_v.at[b]], rows_v.at[b], g_sem.at[b]).wait()
                wb_copy(cnt - 1, b).start()

        for b in range(_NBUF):
            @pl.when(b < cnt)
            def _(b=b):
                wb_copy(0, b).wait()

    return gather_kernel(xT, idx)


def _tc_median_body(n_total, w_ref, xg_ref, x_ref, out_ref, med_ref, slf_ref):
    nb = med_ref.shape[0]
    f = med_ref.shape[1]
    valid = n_total - pl.program_id(0) * nb

    eye_f = (jax.lax.broadcasted_iota(jnp.int32, (f, f), 0) ==
             jax.lax.broadcasted_iota(jnp.int32, (f, f), 1)).astype(jnp.float32)
    slf_ref[...] = jax.lax.dot_general(
        x_ref[0, :, :], eye_f, (((0,), (0,)), ((), ())),
        preferred_element_type=jnp.float32)

    def sub(s, carry):
        row = pl.multiple_of(s * 8, 8)
        v = {}
        for a in range(32):
            v[a] = xg_ref[a, pl.ds(row, 8), :]
        for (a, b, need_min, need_max) in _NET:
            va, vb = v[a], v[b]
            if need_min:
                lo = jnp.minimum(va, vb)
            if need_max:
                v[b] = jnp.maximum(va, vb)
            if need_min:
                v[a] = lo
        slf = slf_ref[pl.ds(row, 8), :]
        med = jnp.minimum(jnp.maximum(slf, v[15]), v[16])
        rid = row + jax.lax.broadcasted_iota(jnp.int32, (8, f), 0)
        med = jnp.where(rid < valid, med, 0.0)
        med_ref[pl.ds(row, 8), :] = med
        return carry

    jax.lax.fori_loop(0, nb // 8, sub, 0)

    w0 = w_ref[0, 0]
    w1 = w_ref[0, 1]
    med = med_ref[...]
    eye = (jax.lax.broadcasted_iota(jnp.int32, (nb, nb), 0) ==
           jax.lax.broadcasted_iota(jnp.int32, (nb, nb), 1)).astype(med.dtype)
    medT = jax.lax.dot_general(med, eye, (((0,), (0,)), ((), ())),
                               preferred_element_type=jnp.float32)
    out_ref[0, :, :] = w0 * x_ref[0, :, :] + w1 * medT


def _tc_median(xg3, x, weight, interpret=False):
    m, n_pad, F = xg3.shape
    B, _, N = x.shape
    NB = _NB
    n_blocks = n_pad // NB
    return pl.pallas_call(
        functools.partial(_tc_median_body, N),
        grid=(n_blocks,),
        in_specs=[
            pl.BlockSpec(memory_space=pltpu.SMEM),
            pl.BlockSpec((m, NB, F), lambda i: (0, i, 0)),
            pl.BlockSpec((1, F, NB), lambda i: (0, 0, i)),
        ],
        out_specs=pl.BlockSpec((1, F, NB), lambda i: (0, 0, i)),
        out_shape=jax.ShapeDtypeStruct((B, F, N), jnp.float32),
        scratch_shapes=[pltpu.VMEM((NB, F), xg3.dtype),
                        pltpu.VMEM((NB, F), jnp.float32)],
        interpret=interpret,
    )(weight, xg3, x)


_NB = 512
_K = 2


def kernel(x, neighbors, weight):
    B, F, N = x.shape
    NB = _NB
    n_blocks = -(-N // NB)
    n_pad = n_blocks * NB
    xT = jnp.pad(x[0].T, ((0, n_pad - N), (0, 0)))
    idx = jnp.pad(neighbors.T.astype(jnp.int32),
                  ((0, 0), (0, n_pad - N)))
    blocks_per_chunk = -(-n_blocks // _K)
    span = blocks_per_chunk * NB
    outs = []
    for k in range(_K):
        lo = k * span
        sz = min(span, n_pad - lo)
        if sz <= 0:
            break
        idx_k = idx[:, lo:lo + sz].reshape(-1)
        xg = _sc_gather(xT, idx_k)
        xg3 = xg.reshape(idx_k.shape[0] // sz, sz, F)
        x_k = jax.lax.dynamic_slice_in_dim(x, lo, min(sz, N - lo), axis=2)
        outs.append(_tc_median(xg3, x_k, weight))
    return jnp.concatenate(outs, axis=2) if len(outs) > 1 else outs[0]
